# Initial kernel scaffold; baseline (speedup 1.0000x reference)
import jax
import jax.numpy as jnp
from jax.experimental import pallas as pl


def kernel(x, bw_w1, bw_b1, bw_w2, bw_b2, bw_w3, bw_b3, bw_w4, bw_b4, gin_w1_0, gin_w1_rest, gin_b1, gin_w2, gin_b2, edge_index, batch):
    raise NotImplementedError("write your pallas kernel here")


# speedup vs baseline: 4.2664x; 4.2664x over previous
import functools

import jax
import jax.numpy as jnp
from jax import lax
from jax.experimental import pallas as pl
from jax.experimental.pallas import tpu as pltpu
from jax.experimental.pallas import tpu_sc as plsc

N = 10000
E = 320000
IN_DIM = 128
HID = 64
EMB = 320
G = 64

BN = 1000
NB = N // BN

_HIGH = lax.Precision.HIGHEST


def _dot(a, b):
    return lax.dot_general(a, b, (((1,), (0,)), ((), ())), precision=_HIGH,
                           preferred_element_type=jnp.float32)


_K = 128
_EPW = E // 16
_NF = _EPW // _K
_TAIL = _EPW - _NF * _K
_NPAD = 10240
_NPW = _NPAD // 16


@functools.cache
def _segsum_call(Dh):
    mesh = plsc.VectorSubcoreMesh(core_axis_name="c", subcore_axis_name="s")

    def body(h_hbm, src_hbm, dst_hbm, out_hbm, acc, sidx, didx, rows,
             sidx_t, didx_t, rows_t, sem):
        cid = lax.axis_index("c")
        sid = lax.axis_index("s")

        @pl.loop(0, _K)
        def _(r):
            @pl.loop(0, Dh, step=16)
            def _(c):
                rows[r, pl.ds(c, 16)] = jnp.zeros((16,), jnp.float32)

        @pl.loop(0, _NPW // _K)
        def _(j):
            pltpu.sync_copy(rows,
                            acc.at[pl.ds(sid * _NPW + j * _K, _K)])

        plsc.subcore_barrier()

        hc = h_hbm.at[cid]

        def chunk(base, si, di, rw):
            pltpu.sync_copy(src_hbm.at[pl.ds(base, si.shape[0])], si)
            pltpu.sync_copy(dst_hbm.at[pl.ds(base, di.shape[0])], di)
            pltpu.async_copy(hc.at[si], rw, sem).wait()
            pltpu.sync_copy(rw, acc.at[di], add=True)

        ebase = sid * _EPW

        @pl.loop(0, _NF)
        def _(i):
            chunk(ebase + i * _K, sidx, didx, rows)

        if _TAIL:
            chunk(ebase + _NF * _K, sidx_t, didx_t, rows_t)

        plsc.subcore_barrier()
        pltpu.sync_copy(acc.at[pl.ds(sid * _NPW, _NPW)],
                        out_hbm.at[cid].at[pl.ds(sid * _NPW, _NPW)])

    return pl.kernel(
        body,
        mesh=mesh,
        compiler_params=pltpu.CompilerParams(use_tc_tiling_on_sc=False),
        out_type=jax.ShapeDtypeStruct((2, _NPAD, Dh), jnp.float32),
        scratch_types=[
            pltpu.VMEM_SHARED((_NPAD, Dh), jnp.float32),
            pltpu.VMEM((_K,), jnp.int32),
            pltpu.VMEM((_K,), jnp.int32),
            pltpu.VMEM((_K, Dh), jnp.float32),
            pltpu.VMEM((_TAIL,), jnp.int32),
            pltpu.VMEM((_TAIL,), jnp.int32),
            pltpu.VMEM((_TAIL, Dh), jnp.float32),
            pltpu.SemaphoreType.DMA,
        ],
    )


def _segsum(hsplit, src, dst):
    return _segsum_call(hsplit.shape[2])(hsplit, src, dst)


def _cnt(batch3):
    def body(b_ref, cnt_ref):
        i = pl.program_id(0)
        b2 = b_ref[0]
        iota = lax.broadcasted_iota(jnp.int32, (G, BN), 0)
        ind = (iota == b2).astype(jnp.float32)

        @pl.when(i == 0)
        def _():
            cnt_ref[...] = jnp.zeros_like(cnt_ref)

        cnt_ref[...] += jnp.sum(ind, axis=1, keepdims=True)

    return pl.pallas_call(
        body, grid=(NB,),
        in_specs=[pl.BlockSpec((1, 1, BN), lambda i: (i, 0, 0))],
        out_specs=pl.BlockSpec((G, 1), lambda i: (0, 0)),
        out_shape=jax.ShapeDtypeStruct((G, 1), jnp.float32))(batch3)


def _pmat(batch3, cnt):
    def body(b_ref, cnt_ref, p_ref):
        b2 = b_ref[0]
        iota = lax.broadcasted_iota(jnp.int32, (G, BN), 0)
        ind = (iota == b2).astype(jnp.float32)
        p_ref[0] = ind / jnp.maximum(cnt_ref[...], 1.0)

    return pl.pallas_call(
        body, grid=(NB,),
        in_specs=[pl.BlockSpec((1, 1, BN), lambda i: (i, 0, 0)),
                  pl.BlockSpec((G, 1), lambda i: (0, 0))],
        out_specs=pl.BlockSpec((1, G, BN), lambda i: (i, 0, 0)),
        out_shape=jax.ShapeDtypeStruct((NB, G, BN), jnp.float32))(batch3, cnt)


def _enc(x, w1, b1, w2, b2, deg):
    def body(x_ref, w1_ref, b1_ref, w2_ref, b2_ref, deg_ref, h_ref, hds_ref):
        h0 = jnp.maximum(_dot(x_ref[...], w1_ref[...]) + b1_ref[...], 0.0)
        h = jnp.maximum(_dot(h0, w2_ref[...]) + b2_ref[...], 0.0)
        h_ref[...] = h
        dinv = lax.rsqrt(jnp.maximum(deg_ref[...], 1.0))
        hd = h * dinv
        hds_ref[0] = hd[:, :EMB // 2]
        hds_ref[1] = hd[:, EMB // 2:]

    return pl.pallas_call(
        body, grid=(NB,),
        in_specs=[pl.BlockSpec((BN, IN_DIM), lambda i: (i, 0)),
                  pl.BlockSpec((IN_DIM, EMB), lambda i: (0, 0)),
                  pl.BlockSpec((1, EMB), lambda i: (0, 0)),
                  pl.BlockSpec((EMB, EMB), lambda i: (0, 0)),
                  pl.BlockSpec((1, EMB), lambda i: (0, 0)),
                  pl.BlockSpec((BN, 1), lambda i: (i, 0))],
        out_specs=[pl.BlockSpec((BN, EMB), lambda i: (i, 0)),
                   pl.BlockSpec((2, BN, EMB // 2), lambda i: (0, i, 0))],
        out_shape=[jax.ShapeDtypeStruct((N, EMB), jnp.float32),
                   jax.ShapeDtypeStruct((2, N, EMB // 2), jnp.float32)],
    )(x, w1, b1, w2, b2, deg)


def _combine(h, t, deg, want_ld):
    def body(h_ref, t_ref, deg_ref, l_ref, *maybe_lds):
        dinv = lax.rsqrt(jnp.maximum(deg_ref[...], 1.0))
        tcat = jnp.concatenate([t_ref[0], t_ref[1]], axis=1)
        L = h_ref[...] - dinv * tcat
        l_ref[...] = L
        if want_ld:
            ld = dinv * L
            maybe_lds[0][0] = ld[:, :EMB // 2]
            maybe_lds[0][1] = ld[:, EMB // 2:]

    out_specs = [pl.BlockSpec((BN, EMB), lambda i: (i, 0))]
    out_shape = [jax.ShapeDtypeStruct((N, EMB), jnp.float32)]
    if want_ld:
        out_specs.append(pl.BlockSpec((2, BN, EMB // 2), lambda i: (0, i, 0)))
        out_shape.append(jax.ShapeDtypeStruct((2, N, EMB // 2), jnp.float32))

    return pl.pallas_call(
        body, grid=(NB,),
        in_specs=[pl.BlockSpec((BN, EMB), lambda i: (i, 0)),
                  pl.BlockSpec((2, BN, EMB // 2), lambda i: (0, i, 0)),
                  pl.BlockSpec((BN, 1), lambda i: (i, 0))],
        out_specs=out_specs, out_shape=out_shape)(h, t, deg)


def _bwfinal(h, l1, l2, wa, wb, wc, b3, pm):
    def body(h_ref, l1_ref, l2_ref, a_ref, b_ref, c_ref, b3_ref, pm_ref,
             acc_ref):
        A = a_ref[...]
        B = b_ref[...]
        C = c_ref[...]
        WB = B - A
        WC = 0.25 * A - 0.5 * B + 0.25 * C
        hf = (_dot(h_ref[...], A) + _dot(l1_ref[...], WB)
              + _dot(l2_ref[...], WC) + b3_ref[...])
        hf = jnp.maximum(hf, 0.0)
        i = pl.program_id(0)

        @pl.when(i == 0)
        def _():
            acc_ref[...] = jnp.zeros_like(acc_ref)

        acc_ref[...] += _dot(pm_ref[0], hf)

    wspec = pl.BlockSpec((EMB, EMB), lambda i: (0, 0))
    return pl.pallas_call(
        body, grid=(NB,),
        in_specs=[pl.BlockSpec((BN, EMB), lambda i: (i, 0)),
                  pl.BlockSpec((BN, EMB), lambda i: (i, 0)),
                  pl.BlockSpec((BN, EMB), lambda i: (i, 0)),
                  wspec, wspec, wspec,
                  pl.BlockSpec((1, EMB), lambda i: (0, 0)),
                  pl.BlockSpec((1, G, BN), lambda i: (i, 0, 0))],
        out_specs=pl.BlockSpec((G, EMB), lambda i: (0, 0)),
        out_shape=jax.ShapeDtypeStruct((G, EMB), jnp.float32),
    )(h, l1, l2, wa, wb, wc, b3, pm)


def _bwhead(pooled, w4, b4, cnt):
    def body(p_ref, w4_ref, b4_ref, cnt_ref, o_ref):
        occ = (cnt_ref[...] > 0.0).astype(jnp.float32)
        o_ref[...] = _dot(p_ref[...], w4_ref[...]) + b4_ref[...] * occ

    return pl.pallas_call(
        body,
        in_specs=[pl.BlockSpec((G, EMB), lambda: (0, 0)),
                  pl.BlockSpec((EMB, EMB), lambda: (0, 0)),
                  pl.BlockSpec((1, EMB), lambda: (0, 0)),
                  pl.BlockSpec((G, 1), lambda: (0, 0))],
        out_specs=pl.BlockSpec((G, EMB), lambda: (0, 0)),
        out_shape=jax.ShapeDtypeStruct((G, EMB), jnp.float32),
    )(pooled, w4, b4, cnt)


def _gin_layer(hp, t, w1, b1, w2, b2, pm, first, last):
    din = IN_DIM if first else HID
    ta = hp.shape
    del ta
    tdh = t.shape[2]

    def body(hp_ref, t_ref, w1_ref, b1_ref, w2_ref, b2_ref, pm_ref,
             h_ref, *rest):
        if first:
            agg = jnp.concatenate([t_ref[0], t_ref[1][:, :IN_DIM - tdh]],
                                  axis=1)
        else:
            agg = jnp.concatenate([t_ref[0], t_ref[1]], axis=1)
        inp = hp_ref[...] + agg
        v = jnp.maximum(_dot(inp, w1_ref[...]) + b1_ref[...], 0.0)
        h = jnp.maximum(_dot(v, w2_ref[...]) + b2_ref[...], 0.0)
        h_ref[...] = h
        if last:
            pool_ref = rest[0]
        else:
            hs_ref, pool_ref = rest
            hs_ref[0] = h[:, :HID // 2]
            hs_ref[1] = h[:, HID // 2:]
        i = pl.program_id(0)

        @pl.when(i == 0)
        def _():
            pool_ref[...] = jnp.zeros_like(pool_ref)

        pool_ref[...] += _dot(pm_ref[0], h)

    out_specs = [pl.BlockSpec((BN, HID), lambda i: (i, 0))]
    out_shape = [jax.ShapeDtypeStruct((N, HID), jnp.float32)]
    if not last:
        out_specs.append(pl.BlockSpec((2, BN, HID // 2), lambda i: (0, i, 0)))
        out_shape.append(jax.ShapeDtypeStruct((2, N, HID // 2), jnp.float32))
    out_specs.append(pl.BlockSpec((G, HID), lambda i: (0, 0)))
    out_shape.append(jax.ShapeDtypeStruct((G, HID), jnp.float32))

    return pl.pallas_call(
        body, grid=(NB,),
        in_specs=[pl.BlockSpec((BN, din), lambda i: (i, 0)),
                  pl.BlockSpec((2, BN, tdh), lambda i: (0, i, 0)),
                  pl.BlockSpec((din, HID), lambda i: (0, 0)),
                  pl.BlockSpec((1, HID), lambda i: (0, 0)),
                  pl.BlockSpec((HID, HID), lambda i: (0, 0)),
                  pl.BlockSpec((1, HID), lambda i: (0, 0)),
                  pl.BlockSpec((1, G, BN), lambda i: (i, 0, 0))],
        out_specs=out_specs, out_shape=out_shape,
    )(hp, t, w1, b1, w2, b2, pm)


def kernel(x, bw_w1, bw_b1, bw_w2, bw_b2, bw_w3, bw_b3, bw_w4, bw_b4,
           gin_w1_0, gin_w1_rest, gin_b1, gin_w2, gin_b2, edge_index, batch):
    src = edge_index[0]
    dst = edge_index[1]
    batch3 = batch.reshape(NB, 1, BN)

    xa = jnp.concatenate(
        [x, jnp.ones((N, 1), jnp.float32), jnp.zeros((N, 31), jnp.float32)],
        axis=1)
    xs = jnp.stack([xa[:, :80], xa[:, 80:]], axis=0)

    agg0 = _segsum(xs, src, dst)
    deg = agg0[1][:, 48:49]

    cnt = _cnt(batch3)
    pm = _pmat(batch3, cnt)

    h, hds = _enc(x, bw_w1, bw_b1.reshape(1, EMB), bw_w2,
                  bw_b2.reshape(1, EMB), deg)
    t1 = _segsum(hds, src, dst)
    l1, l1ds = _combine(h, t1, deg, True)
    t2 = _segsum(l1ds, src, dst)
    (l2,) = _combine(l1, t2, deg, False)
    wa = bw_w3[:EMB]
    wb = bw_w3[EMB:2 * EMB]
    wc = bw_w3[2 * EMB:]
    poolbw = _bwfinal(h, l1, l2, wa, wb, wc, bw_b3.reshape(1, EMB), pm)
    g1 = _bwhead(poolbw, bw_w4, bw_b4.reshape(1, EMB), cnt)

    hp = x
    t = agg0
    pools = []
    for i in range(5):
        w1 = gin_w1_0 if i == 0 else gin_w1_rest[i - 1]
        outs = _gin_layer(hp, t, w1, gin_b1[i].reshape(1, HID),
                          gin_w2[i], gin_b2[i].reshape(1, HID), pm,
                          first=(i == 0), last=(i == 4))
        if i < 4:
            hnew, hsplit, pool_i = outs
            t = _segsum(hsplit, src, dst)
        else:
            hnew, pool_i = outs
        hp = hnew
        pools.append(pool_i)
    g2 = jnp.concatenate(pools, axis=1)

    return (g1, g2)


# speedup vs baseline: 4.2807x; 1.0034x over previous
import functools

import jax
import jax.numpy as jnp
from jax import lax
from jax.experimental import pallas as pl
from jax.experimental.pallas import tpu as pltpu
from jax.experimental.pallas import tpu_sc as plsc

N = 10000
E = 320000
IN_DIM = 128
HID = 64
EMB = 320
G = 64

BN = 1000
NB = N // BN

_HIGH = lax.Precision.HIGHEST


def _dot(a, b):
    return lax.dot_general(a, b, (((1,), (0,)), ((), ())), precision=_HIGH,
                           preferred_element_type=jnp.float32)


_EPAD = 327680
_NPAD = 10240
_NPW = _NPAD // 16
_SINK = N


def _chunk_k(Dh):
    return 64 if Dh >= 128 else 128


@functools.cache
def _segsum_call(Dh):
    K = _chunk_k(Dh)
    CPS = _EPAD // K // 16
    mesh = plsc.VectorSubcoreMesh(core_axis_name="c", subcore_axis_name="s")

    def body(h_hbm, src_hbm, dst_hbm, out_hbm, acc, si0, si1, di0, di1,
             rows0, rows1, sg0, sg1, ss0, ss1, sd0, sd1):
        cid = lax.axis_index("c")
        sid = lax.axis_index("s")
        base = sid * CPS

        @pl.loop(0, K)
        def _(r):
            @pl.loop(0, Dh, step=16)
            def _(c):
                rows0[r, pl.ds(c, 16)] = jnp.zeros((16,), jnp.float32)

        @pl.loop(0, _NPW // K)
        def _(j):
            pltpu.sync_copy(rows0, acc.at[pl.ds(sid * _NPW + j * K, K)])

        plsc.subcore_barrier()

        hc = h_hbm.at[cid]

        def idx_load(i, si, di, ss, sd):
            pltpu.make_async_copy(src_hbm.at[base + i], si, ss).start()
            pltpu.make_async_copy(dst_hbm.at[base + i], di, sd).start()

        def idx_wait(si, di, ss, sd):
            pltpu.make_async_copy(src_hbm.at[0], si, ss).wait()
            pltpu.make_async_copy(dst_hbm.at[0], di, sd).wait()

        def gather(si, rw, sg):
            return pltpu.make_async_copy(hc.at[si], rw, sg)

        idx_load(0, si0, di0, ss0, sd0)
        idx_wait(si0, di0, ss0, sd0)
        gather(si0, rows0, sg0).start()
        idx_load(1, si1, di1, ss1, sd1)

        @pl.loop(0, CPS // 2)
        def _(t):
            i0 = 2 * t
            idx_wait(si1, di1, ss1, sd1)
            gather(si1, rows1, sg1).start()
            gather(si0, rows0, sg0).wait()
            pltpu.sync_copy(rows0, acc.at[di0], add=True)

            @pl.when(i0 + 2 < CPS)
            def _():
                idx_load(i0 + 2, si0, di0, ss0, sd0)
                idx_wait(si0, di0, ss0, sd0)
                gather(si0, rows0, sg0).start()

            gather(si1, rows1, sg1).wait()
            pltpu.sync_copy(rows1, acc.at[di1], add=True)

            @pl.when(i0 + 3 < CPS)
            def _():
                idx_load(i0 + 3, si1, di1, ss1, sd1)

        plsc.subcore_barrier()
        pltpu.sync_copy(acc.at[pl.ds(sid * _NPW, _NPW)],
                        out_hbm.at[cid].at[pl.ds(sid * _NPW, _NPW)])

    return pl.kernel(
        body,
        mesh=mesh,
        compiler_params=pltpu.CompilerParams(use_tc_tiling_on_sc=False),
        out_type=jax.ShapeDtypeStruct((2, _NPAD, Dh), jnp.float32),
        scratch_types=[
            pltpu.VMEM_SHARED((_NPAD, Dh), jnp.float32),
            pltpu.VMEM((K,), jnp.int32),
            pltpu.VMEM((K,), jnp.int32),
            pltpu.VMEM((K,), jnp.int32),
            pltpu.VMEM((K,), jnp.int32),
            pltpu.VMEM((K, Dh), jnp.float32),
            pltpu.VMEM((K, Dh), jnp.float32),
            pltpu.SemaphoreType.DMA,
            pltpu.SemaphoreType.DMA,
            pltpu.SemaphoreType.DMA,
            pltpu.SemaphoreType.DMA,
            pltpu.SemaphoreType.DMA,
            pltpu.SemaphoreType.DMA,
        ],
    )


def _segsum(hsplit, src_pad, dst_pad):
    Dh = hsplit.shape[2]
    K = _chunk_k(Dh)
    return _segsum_call(Dh)(hsplit, src_pad.reshape(-1, K),
                            dst_pad.reshape(-1, K))


def _cnt(batch3):
    def body(b_ref, cnt_ref):
        i = pl.program_id(0)
        b2 = b_ref[0]
        iota = lax.broadcasted_iota(jnp.int32, (G, BN), 0)
        ind = (iota == b2).astype(jnp.float32)

        @pl.when(i == 0)
        def _():
            cnt_ref[...] = jnp.zeros_like(cnt_ref)

        cnt_ref[...] += jnp.sum(ind, axis=1, keepdims=True)

    return pl.pallas_call(
        body, grid=(NB,),
        in_specs=[pl.BlockSpec((1, 1, BN), lambda i: (i, 0, 0))],
        out_specs=pl.BlockSpec((G, 1), lambda i: (0, 0)),
        out_shape=jax.ShapeDtypeStruct((G, 1), jnp.float32))(batch3)


def _pmat(batch3, cnt):
    def body(b_ref, cnt_ref, p_ref):
        b2 = b_ref[0]
        iota = lax.broadcasted_iota(jnp.int32, (G, BN), 0)
        ind = (iota == b2).astype(jnp.float32)
        p_ref[0] = ind / jnp.maximum(cnt_ref[...], 1.0)

    return pl.pallas_call(
        body, grid=(NB,),
        in_specs=[pl.BlockSpec((1, 1, BN), lambda i: (i, 0, 0)),
                  pl.BlockSpec((G, 1), lambda i: (0, 0))],
        out_specs=pl.BlockSpec((1, G, BN), lambda i: (i, 0, 0)),
        out_shape=jax.ShapeDtypeStruct((NB, G, BN), jnp.float32))(batch3, cnt)


def _enc(x, w1, b1, w2, b2, deg):
    def body(x_ref, w1_ref, b1_ref, w2_ref, b2_ref, deg_ref, h_ref, hds_ref):
        h0 = jnp.maximum(_dot(x_ref[...], w1_ref[...]) + b1_ref[...], 0.0)
        h = jnp.maximum(_dot(h0, w2_ref[...]) + b2_ref[...], 0.0)
        h_ref[...] = h
        dinv = lax.rsqrt(jnp.maximum(deg_ref[...], 1.0))
        hd = h * dinv
        hds_ref[0] = hd[:, :EMB // 2]
        hds_ref[1] = hd[:, EMB // 2:]

    return pl.pallas_call(
        body, grid=(NB,),
        in_specs=[pl.BlockSpec((BN, IN_DIM), lambda i: (i, 0)),
                  pl.BlockSpec((IN_DIM, EMB), lambda i: (0, 0)),
                  pl.BlockSpec((1, EMB), lambda i: (0, 0)),
                  pl.BlockSpec((EMB, EMB), lambda i: (0, 0)),
                  pl.BlockSpec((1, EMB), lambda i: (0, 0)),
                  pl.BlockSpec((BN, 1), lambda i: (i, 0))],
        out_specs=[pl.BlockSpec((BN, EMB), lambda i: (i, 0)),
                   pl.BlockSpec((2, BN, EMB // 2), lambda i: (0, i, 0))],
        out_shape=[jax.ShapeDtypeStruct((N, EMB), jnp.float32),
                   jax.ShapeDtypeStruct((2, N, EMB // 2), jnp.float32)],
    )(x, w1, b1, w2, b2, deg)


def _combine(h, t, deg, want_ld):
    def body(h_ref, t_ref, deg_ref, l_ref, *maybe_lds):
        dinv = lax.rsqrt(jnp.maximum(deg_ref[...], 1.0))
        tcat = jnp.concatenate([t_ref[0], t_ref[1]], axis=1)
        L = h_ref[...] - dinv * tcat
        l_ref[...] = L
        if want_ld:
            ld = dinv * L
            maybe_lds[0][0] = ld[:, :EMB // 2]
            maybe_lds[0][1] = ld[:, EMB // 2:]

    out_specs = [pl.BlockSpec((BN, EMB), lambda i: (i, 0))]
    out_shape = [jax.ShapeDtypeStruct((N, EMB), jnp.float32)]
    if want_ld:
        out_specs.append(pl.BlockSpec((2, BN, EMB // 2), lambda i: (0, i, 0)))
        out_shape.append(jax.ShapeDtypeStruct((2, N, EMB // 2), jnp.float32))

    return pl.pallas_call(
        body, grid=(NB,),
        in_specs=[pl.BlockSpec((BN, EMB), lambda i: (i, 0)),
                  pl.BlockSpec((2, BN, EMB // 2), lambda i: (0, i, 0)),
                  pl.BlockSpec((BN, 1), lambda i: (i, 0))],
        out_specs=out_specs, out_shape=out_shape)(h, t, deg)


def _bwfinal(h, l1, l2, wa, wb, wc, b3, pm):
    def body(h_ref, l1_ref, l2_ref, a_ref, b_ref, c_ref, b3_ref, pm_ref,
             acc_ref):
        A = a_ref[...]
        B = b_ref[...]
        C = c_ref[...]
        WB = B - A
        WC = 0.25 * A - 0.5 * B + 0.25 * C
        hf = (_dot(h_ref[...], A) + _dot(l1_ref[...], WB)
              + _dot(l2_ref[...], WC) + b3_ref[...])
        hf = jnp.maximum(hf, 0.0)
        i = pl.program_id(0)

        @pl.when(i == 0)
        def _():
            acc_ref[...] = jnp.zeros_like(acc_ref)

        acc_ref[...] += _dot(pm_ref[0], hf)

    wspec = pl.BlockSpec((EMB, EMB), lambda i: (0, 0))
    return pl.pallas_call(
        body, grid=(NB,),
        in_specs=[pl.BlockSpec((BN, EMB), lambda i: (i, 0)),
                  pl.BlockSpec((BN, EMB), lambda i: (i, 0)),
                  pl.BlockSpec((BN, EMB), lambda i: (i, 0)),
                  wspec, wspec, wspec,
                  pl.BlockSpec((1, EMB), lambda i: (0, 0)),
                  pl.BlockSpec((1, G, BN), lambda i: (i, 0, 0))],
        out_specs=pl.BlockSpec((G, EMB), lambda i: (0, 0)),
        out_shape=jax.ShapeDtypeStruct((G, EMB), jnp.float32),
    )(h, l1, l2, wa, wb, wc, b3, pm)


def _bwhead(pooled, w4, b4, cnt):
    def body(p_ref, w4_ref, b4_ref, cnt_ref, o_ref):
        occ = (cnt_ref[...] > 0.0).astype(jnp.float32)
        o_ref[...] = _dot(p_ref[...], w4_ref[...]) + b4_ref[...] * occ

    return pl.pallas_call(
        body,
        in_specs=[pl.BlockSpec((G, EMB), lambda: (0, 0)),
                  pl.BlockSpec((EMB, EMB), lambda: (0, 0)),
                  pl.BlockSpec((1, EMB), lambda: (0, 0)),
                  pl.BlockSpec((G, 1), lambda: (0, 0))],
        out_specs=pl.BlockSpec((G, EMB), lambda: (0, 0)),
        out_shape=jax.ShapeDtypeStruct((G, EMB), jnp.float32),
    )(pooled, w4, b4, cnt)


def _gin_layer(hp, t, w1, b1, w2, b2, pm, first, last):
    din = IN_DIM if first else HID
    ta = hp.shape
    del ta
    tdh = t.shape[2]

    def body(hp_ref, t_ref, w1_ref, b1_ref, w2_ref, b2_ref, pm_ref,
             h_ref, *rest):
        if first:
            agg = jnp.concatenate([t_ref[0], t_ref[1][:, :IN_DIM - tdh]],
                                  axis=1)
        else:
            agg = jnp.concatenate([t_ref[0], t_ref[1]], axis=1)
        inp = hp_ref[...] + agg
        v = jnp.maximum(_dot(inp, w1_ref[...]) + b1_ref[...], 0.0)
        h = jnp.maximum(_dot(v, w2_ref[...]) + b2_ref[...], 0.0)
        h_ref[...] = h
        if last:
            pool_ref = rest[0]
        else:
            hs_ref, pool_ref = rest
            hs_ref[0] = h[:, :HID // 2]
            hs_ref[1] = h[:, HID // 2:]
        i = pl.program_id(0)

        @pl.when(i == 0)
        def _():
            pool_ref[...] = jnp.zeros_like(pool_ref)

        pool_ref[...] += _dot(pm_ref[0], h)

    out_specs = [pl.BlockSpec((BN, HID), lambda i: (i, 0))]
    out_shape = [jax.ShapeDtypeStruct((N, HID), jnp.float32)]
    if not last:
        out_specs.append(pl.BlockSpec((2, BN, HID // 2), lambda i: (0, i, 0)))
        out_shape.append(jax.ShapeDtypeStruct((2, N, HID // 2), jnp.float32))
    out_specs.append(pl.BlockSpec((G, HID), lambda i: (0, 0)))
    out_shape.append(jax.ShapeDtypeStruct((G, HID), jnp.float32))

    return pl.pallas_call(
        body, grid=(NB,),
        in_specs=[pl.BlockSpec((BN, din), lambda i: (i, 0)),
                  pl.BlockSpec((2, BN, tdh), lambda i: (0, i, 0)),
                  pl.BlockSpec((din, HID), lambda i: (0, 0)),
                  pl.BlockSpec((1, HID), lambda i: (0, 0)),
                  pl.BlockSpec((HID, HID), lambda i: (0, 0)),
                  pl.BlockSpec((1, HID), lambda i: (0, 0)),
                  pl.BlockSpec((1, G, BN), lambda i: (i, 0, 0))],
        out_specs=out_specs, out_shape=out_shape,
    )(hp, t, w1, b1, w2, b2, pm)


def kernel(x, bw_w1, bw_b1, bw_w2, bw_b2, bw_w3, bw_b3, bw_w4, bw_b4,
           gin_w1_0, gin_w1_rest, gin_b1, gin_w2, gin_b2, edge_index, batch):
    src = edge_index[0]
    dst = edge_index[1]
    pad = _EPAD - E
    src2d = jnp.concatenate([src, jnp.zeros((pad,), jnp.int32)])
    dst2d = jnp.concatenate([dst, jnp.full((pad,), _SINK, jnp.int32)])
    batch3 = batch.reshape(NB, 1, BN)

    xa = jnp.concatenate(
        [x, jnp.ones((N, 1), jnp.float32), jnp.zeros((N, 31), jnp.float32)],
        axis=1)
    xs = jnp.stack([xa[:, :80], xa[:, 80:]], axis=0)

    agg0 = _segsum(xs, src2d, dst2d)
    deg = agg0[1][:, 48:49]

    cnt = _cnt(batch3)
    pm = _pmat(batch3, cnt)

    h, hds = _enc(x, bw_w1, bw_b1.reshape(1, EMB), bw_w2,
                  bw_b2.reshape(1, EMB), deg)
    t1 = _segsum(hds, src2d, dst2d)
    l1, l1ds = _combine(h, t1, deg, True)
    t2 = _segsum(l1ds, src2d, dst2d)
    (l2,) = _combine(l1, t2, deg, False)
    wa = bw_w3[:EMB]
    wb = bw_w3[EMB:2 * EMB]
    wc = bw_w3[2 * EMB:]
    poolbw = _bwfinal(h, l1, l2, wa, wb, wc, bw_b3.reshape(1, EMB), pm)
    g1 = _bwhead(poolbw, bw_w4, bw_b4.reshape(1, EMB), cnt)

    hp = x
    t = agg0
    pools = []
    for i in range(5):
        w1 = gin_w1_0 if i == 0 else gin_w1_rest[i - 1]
        outs = _gin_layer(hp, t, w1, gin_b1[i].reshape(1, HID),
                          gin_w2[i], gin_b2[i].reshape(1, HID), pm,
                          first=(i == 0), last=(i == 4))
        if i < 4:
            hnew, hsplit, pool_i = outs
            t = _segsum(hsplit, src2d, dst2d)
        else:
            hnew, pool_i = outs
        hp = hnew
        pools.append(pool_i)
    g2 = jnp.concatenate(pools, axis=1)

    return (g1, g2)
